# baseline (speedup 1.0000x reference)
import functools

import jax
import jax.numpy as jnp
from jax import lax
from jax.experimental import pallas as pl
from jax.experimental.pallas import tpu as pltpu
from jax.experimental.pallas import tpu_sc as plsc

B = 16384
D = 1024
POS_WEIGHT = 2.0
EPS = 1e-8
K = 20
ROWS_PER_BLOCK = 512
NUM_BLOCKS = B // ROWS_PER_BLOCK

NSUB = 16
CHUNK = B // NSUB
NV = CHUNK // 16
SENTINEL = -3.0


def _sim_kernel(a_ref, b_ref, o_ref):
    a = a_ref[...]
    b = b_ref[...]
    num = jnp.sum(a * b, axis=1, keepdims=True)
    na = jnp.sqrt(jnp.sum(a * a, axis=1, keepdims=True))
    nb = jnp.sqrt(jnp.sum(b * b, axis=1, keepdims=True))
    o_ref[...] = num / jnp.maximum(na * nb, EPS)


def _rev(x):
    return lax.rev(x, dimensions=(0,))


def _vsort(x):
    return plsc.sort_key_val(x, x)[0]


def _merge_full(a, b):
    rb = _rev(b)
    lo = jnp.minimum(a, rb)
    hi = jnp.maximum(a, rb)
    return _vsort(lo), _vsort(hi)


def _merge_top(a_pair, b_pair):
    a_lo, a_hi = a_pair
    b_lo, b_hi = b_pair
    c0 = jnp.maximum(a_lo, _rev(b_hi))
    c1 = jnp.maximum(a_hi, _rev(b_lo))
    lo = jnp.minimum(c0, c1)
    hi = jnp.maximum(c0, c1)
    return _vsort(lo), _vsort(hi)


def _top32(vregs):
    runs = [_vsort(v) for v in vregs]
    pairs = [_merge_full(runs[2 * i], runs[2 * i + 1]) for i in range(len(runs) // 2)]
    while len(pairs) > 1:
        pairs = [_merge_top(pairs[2 * i], pairs[2 * i + 1]) for i in range(len(pairs) // 2)]
    return pairs[0]


def _sc_select_kernel(sim_hbm, lab_hbm, out_hbm, sim_v, lab_v, buf_v, gat_v, stage_sh, out_v):
    cid = lax.axis_index("c")
    sid = lax.axis_index("s")

    @pl.when(cid == 0)
    def _core0():
        base = sid * CHUNK
        pltpu.sync_copy(sim_hbm.at[pl.ds(base, CHUNK)], sim_v)
        pltpu.sync_copy(lab_hbm.at[pl.ds(base, CHUNK)], lab_v)
        negs, poss = [], []
        for i in range(NV):
            s = sim_v[pl.ds(i * 16, 16)]
            l = lab_v[pl.ds(i * 16, 16)]
            negs.append(jnp.where(l == 0.0, s, SENTINEL))
            poss.append(jnp.where(l == 0.0, SENTINEL, -s))
        n_lo, n_hi = _top32(negs)
        p_lo, p_hi = _top32(poss)
        buf_v[pl.ds(0, 16)] = n_lo
        buf_v[pl.ds(16, 16)] = n_hi
        buf_v[pl.ds(32, 16)] = p_lo
        buf_v[pl.ds(48, 16)] = p_hi
        pltpu.sync_copy(buf_v, stage_sh.at[pl.ds(sid * 64, 64)])
        plsc.subcore_barrier()

        @pl.when(sid == 0)
        def _merge():
            pltpu.sync_copy(stage_sh, gat_v)
            neg_pairs, pos_pairs = [], []
            for r in range(NSUB):
                o = r * 64
                neg_pairs.append((gat_v[pl.ds(o, 16)], gat_v[pl.ds(o + 16, 16)]))
                pos_pairs.append((gat_v[pl.ds(o + 32, 16)], gat_v[pl.ds(o + 48, 16)]))
            while len(neg_pairs) > 1:
                neg_pairs = [_merge_top(neg_pairs[2 * i], neg_pairs[2 * i + 1])
                             for i in range(len(neg_pairs) // 2)]
                pos_pairs = [_merge_top(pos_pairs[2 * i], pos_pairs[2 * i + 1])
                             for i in range(len(pos_pairs) // 2)]
            n_lo, n_hi = neg_pairs[0]
            p_lo, p_hi = pos_pairs[0]
            lane = lax.iota(jnp.int32, 16)
            keep = lane >= 12
            zero = jnp.zeros((16,), jnp.float32)
            tot_n = (jnp.sum(jnp.where(keep, n_lo * n_lo, zero))
                     + jnp.sum(n_hi * n_hi))
            pl1 = p_lo + 1.0
            ph1 = p_hi + 1.0
            tot_p = (jnp.sum(jnp.where(keep, pl1 * pl1, zero))
                     + jnp.sum(ph1 * ph1))
            loss = (tot_n + tot_p) * (1.0 / (2 * K))
            out_v[...] = jnp.full((16,), 0.0, jnp.float32) + loss
            pltpu.sync_copy(out_v, out_hbm)


_sc_select = functools.partial(
    pl.kernel,
    mesh=plsc.VectorSubcoreMesh(core_axis_name="c", subcore_axis_name="s"),
    out_type=jax.ShapeDtypeStruct((16,), jnp.float32),
    scratch_types=[
        pltpu.VMEM((CHUNK,), jnp.float32),
        pltpu.VMEM((CHUNK,), jnp.float32),
        pltpu.VMEM((64,), jnp.float32),
        pltpu.VMEM((NSUB * 64,), jnp.float32),
        pltpu.VMEM_SHARED((NSUB * 64,), jnp.float32),
        pltpu.VMEM((16,), jnp.float32),
    ],
    compiler_params=pltpu.CompilerParams(needs_layout_passes=False),
)(_sc_select_kernel)


def kernel(sample_1, sample_2, labels, original_target):
    sim = pl.pallas_call(
        _sim_kernel,
        grid=(NUM_BLOCKS,),
        in_specs=[
            pl.BlockSpec((ROWS_PER_BLOCK, D), lambda i: (i, 0)),
            pl.BlockSpec((ROWS_PER_BLOCK, D), lambda i: (i, 0)),
        ],
        out_specs=pl.BlockSpec((ROWS_PER_BLOCK, 1), lambda i: (i, 0)),
        out_shape=jax.ShapeDtypeStruct((B, 1), jnp.float32),
    )(sample_1, sample_2)

    out = _sc_select(sim.reshape(B), labels)
    weight = (POS_WEIGHT - 1.0) * jnp.float32(original_target) + 1.0
    return out[0] * weight


# speedup vs baseline: 1.5199x; 1.5199x over previous
import jax
import jax.numpy as jnp
from jax import lax
from jax.experimental import pallas as pl
from jax.experimental.pallas import tpu as pltpu

B = 16384
D = 1024
POS_WEIGHT = 2.0
EPS = 1e-8
K = 20
ROWS_PER_BLOCK = 1024
NUM_BLOCKS = B // ROWS_PER_BLOCK
SEL_ROWS = 128
SEL_COLS = B // SEL_ROWS
BIG = 1 << 30
TILES = ROWS_PER_BLOCK // SEL_COLS


def _fused_kernel(a_ref, b_ref, lab_ref, o_ref, sim_acc):
    i = pl.program_id(0)
    a = a_ref[...]
    b = b_ref[...]
    num = jnp.sum(a * b, axis=1, keepdims=True)
    na = jnp.sqrt(jnp.sum(a * a, axis=1, keepdims=True))
    nb = jnp.sqrt(jnp.sum(b * b, axis=1, keepdims=True))
    sim = num / jnp.maximum(na * nb, EPS)
    sim_acc[pl.ds(TILES * i, TILES), :] = sim.reshape(TILES, SEL_COLS)

    @pl.when(i == NUM_BLOCKS - 1)
    def _select():
        simf = sim_acc[...]
        lab = lab_ref[...]
        neg = jnp.where(lab == 0.0, simf, -3.0)
        pos = jnp.where(lab == 0.0, 3.0, simf)
        r = lax.broadcasted_iota(jnp.int32, (SEL_ROWS, SEL_COLS), 0)
        c = lax.broadcasted_iota(jnp.int32, (SEL_ROWS, SEL_COLS), 1)
        flat = r * SEL_COLS + c

        def body(_, carry):
            vn, vp, tot_n, tot_p = carry
            mn = jnp.max(vn)
            mp = jnp.min(vp)
            sel_n = jnp.min(jnp.where(vn == mn, flat, BIG))
            sel_p = jnp.min(jnp.where(vp == mp, flat, BIG))
            vn = jnp.where(flat == sel_n, -3.0, vn)
            vp = jnp.where(flat == sel_p, 3.0, vp)
            dp = mp - 1.0
            return vn, vp, tot_n + mn * mn, tot_p + dp * dp

        _, _, tot_n, tot_p = lax.fori_loop(
            0, K, body, (neg, pos, jnp.float32(0.0), jnp.float32(0.0))
        )
        o_ref[...] = jnp.broadcast_to((tot_n + tot_p) * (1.0 / (2 * K)), (1, 1))


def kernel(sample_1, sample_2, labels, original_target):
    lab2d = labels.reshape(SEL_ROWS, SEL_COLS)
    out = pl.pallas_call(
        _fused_kernel,
        grid=(NUM_BLOCKS,),
        in_specs=[
            pl.BlockSpec((ROWS_PER_BLOCK, D), lambda i: (i, 0)),
            pl.BlockSpec((ROWS_PER_BLOCK, D), lambda i: (i, 0)),
            pl.BlockSpec((SEL_ROWS, SEL_COLS), lambda i: (0, 0)),
        ],
        out_specs=pl.BlockSpec((1, 1), lambda i: (0, 0)),
        out_shape=jax.ShapeDtypeStruct((1, 1), jnp.float32),
        scratch_shapes=[pltpu.VMEM((SEL_ROWS, SEL_COLS), jnp.float32)],
    )(sample_1, sample_2, lab2d)

    weight = (POS_WEIGHT - 1.0) * jnp.float32(original_target) + 1.0
    return out[0, 0] * weight


# speedup vs baseline: 1.5312x; 1.0074x over previous
import jax
import jax.numpy as jnp
from jax import lax
from jax.experimental import pallas as pl
from jax.experimental.pallas import tpu as pltpu

B = 16384
D = 1024
POS_WEIGHT = 2.0
EPS = 1e-8
K = 20
ROWS_PER_BLOCK = 1024
NUM_BLOCKS = B // ROWS_PER_BLOCK
SEL_ROWS = 128
SEL_COLS = B // SEL_ROWS
BIG = 1 << 30
TILES = ROWS_PER_BLOCK // SEL_COLS


def _fused_kernel(a_ref, b_ref, lab_ref, o_ref, sim_acc):
    i = pl.program_id(0)
    a = a_ref[...]
    b = b_ref[...]
    num = jnp.sum(a * b, axis=1, keepdims=True)
    na = jnp.sqrt(jnp.sum(a * a, axis=1, keepdims=True))
    nb = jnp.sqrt(jnp.sum(b * b, axis=1, keepdims=True))
    sim = num / jnp.maximum(na * nb, EPS)
    sim_acc[pl.ds(TILES * i, TILES), :] = sim.reshape(TILES, SEL_COLS)

    @pl.when(i == NUM_BLOCKS - 1)
    def _select():
        simf = sim_acc[...]
        lab = lab_ref[...]
        neg = jnp.where(lab == 0.0, simf, -3.0)
        pos = jnp.where(lab == 0.0, 3.0, simf)
        r = lax.broadcasted_iota(jnp.int32, (SEL_ROWS, SEL_COLS), 0)
        c = lax.broadcasted_iota(jnp.int32, (SEL_ROWS, SEL_COLS), 1)
        flat = r * SEL_COLS + c

        vn, vp = neg, pos
        tot_n = jnp.float32(0.0)
        tot_p = jnp.float32(0.0)
        for _ in range(K):
            mn = jnp.max(vn)
            mp = jnp.min(vp)
            sel_n = jnp.min(jnp.where(vn == mn, flat, BIG))
            sel_p = jnp.min(jnp.where(vp == mp, flat, BIG))
            vn = jnp.where(flat == sel_n, -3.0, vn)
            vp = jnp.where(flat == sel_p, 3.0, vp)
            dp = mp - 1.0
            tot_n = tot_n + mn * mn
            tot_p = tot_p + dp * dp
        o_ref[...] = jnp.broadcast_to((tot_n + tot_p) * (1.0 / (2 * K)), (1, 1))


def kernel(sample_1, sample_2, labels, original_target):
    lab2d = labels.reshape(SEL_ROWS, SEL_COLS)
    out = pl.pallas_call(
        _fused_kernel,
        grid=(NUM_BLOCKS,),
        in_specs=[
            pl.BlockSpec((ROWS_PER_BLOCK, D), lambda i: (i, 0)),
            pl.BlockSpec((ROWS_PER_BLOCK, D), lambda i: (i, 0)),
            pl.BlockSpec((SEL_ROWS, SEL_COLS), lambda i: (0, 0)),
        ],
        out_specs=pl.BlockSpec((1, 1), lambda i: (0, 0)),
        out_shape=jax.ShapeDtypeStruct((1, 1), jnp.float32),
        scratch_shapes=[pltpu.VMEM((SEL_ROWS, SEL_COLS), jnp.float32)],
    )(sample_1, sample_2, lab2d)

    weight = (POS_WEIGHT - 1.0) * jnp.float32(original_target) + 1.0
    return out[0, 0] * weight
